# baseline (speedup 1.0000x reference)
import math

import jax
import jax.numpy as jnp
from jax import lax
from jax.experimental import pallas as pl
from jax.experimental.pallas import tpu as pltpu
from jax.experimental.pallas import tpu_sc as plsc

DIM = 64
LANES = 16
NC, NS = 2, 16
NW = NC * NS
SUB = 128
NSUB = 1
CHUNK = SUB * NSUB
NGRP = CHUNK // LANES
SCALE = math.sqrt(DIM)


def _gather_start(table_hbm, idx_v, c, idxs, rows, gsem):
    for k in range(NGRP):
        vec = idx_v[pl.ds(c * CHUNK + k * LANES, LANES)]
        j, r = divmod(k, SUB // LANES)
        idxs[j, pl.ds(r * LANES, LANES)] = lax.shift_right_logical(vec, 2)
    for j in range(NSUB):
        pltpu.async_copy(
            table_hbm.at[idxs.at[j]], rows.at[pl.ds(j * SUB, SUB)], gsem
        )


def _gather_wait(table_hbm, rows, gsem):
    pltpu.make_async_copy(table_hbm.at[pl.ds(0, CHUNK)], rows, gsem).wait()


def _extract(idx_v, c, rows, ob):
    lane = lax.iota(jnp.int32, LANES)

    @plsc.parallel_loop(0, CHUNK, step=1, unroll=4)
    def _row(k):
        par = plsc.load_gather(idx_v, [lane * 0 + (c * CHUNK + k)]) & 3
        row_splat = lane * 0 + k
        col0 = par * DIM + lane
        for j in range(DIM // LANES):
            v = plsc.load_gather(rows, [row_splat, col0 + j * LANES])
            ob[k, pl.ds(j * LANES, LANES)] = v * SCALE


def _emb_body(x_hbm, table_hbm, out_hbm, idx_v, idxs0, idxs1,
              rows0, rows1, ob0, ob1, gsem0, gsem1, ssem0, ssem1):
    wid = lax.axis_index("s") * NC + lax.axis_index("c")
    nchunk = out_hbm.shape[1]
    nloop = nchunk // 2

    pltpu.sync_copy(x_hbm.at[wid], idx_v)
    _gather_start(table_hbm, idx_v, 0, idxs0, rows0, gsem0)

    def pair(i, carry):
        c0 = 2 * i

        @pl.when(i > 0)
        def _():
            pltpu.make_async_copy(ob1, out_hbm.at[wid, c0], ssem1).wait()

        _gather_start(table_hbm, idx_v, c0 + 1, idxs1, rows1, gsem1)
        _gather_wait(table_hbm, rows0, gsem0)

        @pl.when(i > 0)
        def _():
            pltpu.make_async_copy(ob0, out_hbm.at[wid, c0], ssem0).wait()

        _extract(idx_v, c0, rows0, ob0)
        pltpu.async_copy(ob0, out_hbm.at[wid, c0], ssem0)

        @pl.when(i < nloop - 1)
        def _():
            _gather_start(table_hbm, idx_v, c0 + 2, idxs0, rows0, gsem0)

        _gather_wait(table_hbm, rows1, gsem1)
        _extract(idx_v, c0 + 1, rows1, ob1)
        pltpu.async_copy(ob1, out_hbm.at[wid, c0 + 1], ssem1)
        return carry

    lax.fori_loop(0, nloop, pair, jnp.int32(0))
    pltpu.make_async_copy(ob0, out_hbm.at[wid, nchunk - 2], ssem0).wait()
    pltpu.make_async_copy(ob1, out_hbm.at[wid, nchunk - 1], ssem1).wait()


@jax.jit
def kernel(x, table):
    b, l = x.shape
    voc, dim = table.shape
    total = b * l
    nchunk = total // (NW * CHUNK)
    xr = x.astype(jnp.int32).reshape(NW, nchunk * CHUNK)
    t2 = table.reshape(voc // 4, 4 * dim)
    mesh = plsc.VectorSubcoreMesh(
        core_axis_name="c", subcore_axis_name="s",
        num_cores=NC, num_subcores=NS,
    )
    out = pl.kernel(
        _emb_body,
        out_type=jax.ShapeDtypeStruct((NW, nchunk, CHUNK, DIM), jnp.float32),
        mesh=mesh,
        compiler_params=pltpu.CompilerParams(needs_layout_passes=False),
        scratch_types=[
            pltpu.VMEM((nchunk * CHUNK,), jnp.int32),
            pltpu.VMEM((NSUB, SUB), jnp.int32),
            pltpu.VMEM((NSUB, SUB), jnp.int32),
            pltpu.VMEM((CHUNK, 4 * DIM), jnp.float32),
            pltpu.VMEM((CHUNK, 4 * DIM), jnp.float32),
            pltpu.VMEM((CHUNK, DIM), jnp.float32),
            pltpu.VMEM((CHUNK, DIM), jnp.float32),
            pltpu.SemaphoreType.DMA,
            pltpu.SemaphoreType.DMA,
            pltpu.SemaphoreType.DMA,
            pltpu.SemaphoreType.DMA,
        ],
    )(xr, t2)
    return out.reshape(b, l, DIM)


# speedup vs baseline: 1.1417x; 1.1417x over previous
import math

import jax
import jax.numpy as jnp
from jax import lax
from jax.experimental import pallas as pl
from jax.experimental.pallas import tpu as pltpu
from jax.experimental.pallas import tpu_sc as plsc

DIM = 64
LANES = 16
NC, NS = 2, 16
NW = NC * NS
SUB = 128
NSUB = 1
CHUNK = SUB * NSUB
NGRP = CHUNK // LANES
SCALE = math.sqrt(DIM)


def _gather_start(table_hbm, idx_v, c, idxs, rows, gsem):
    for k in range(NGRP):
        vec = idx_v[pl.ds(c * CHUNK + k * LANES, LANES)]
        j, r = divmod(k, SUB // LANES)
        idxs[j, pl.ds(r * LANES, LANES)] = lax.shift_right_logical(vec, 1)
    for j in range(NSUB):
        pltpu.async_copy(
            table_hbm.at[idxs.at[j]], rows.at[pl.ds(j * SUB, SUB)], gsem
        )


def _gather_wait(table_hbm, rows, gsem):
    pltpu.make_async_copy(table_hbm.at[pl.ds(0, CHUNK)], rows, gsem).wait()


def _extract(idx_v, c, rows, ob):
    lane = lax.iota(jnp.int32, LANES)

    @plsc.parallel_loop(0, CHUNK, step=1, unroll=4)
    def _row(k):
        par = plsc.load_gather(idx_v, [lane * 0 + (c * CHUNK + k)]) & 1
        row_splat = lane * 0 + k
        col0 = par * DIM + lane
        for j in range(DIM // LANES):
            v = plsc.load_gather(rows, [row_splat, col0 + j * LANES])
            ob[k, pl.ds(j * LANES, LANES)] = v * SCALE


def _emb_body(x_hbm, table_hbm, out_hbm, idx_v, idxs0, idxs1,
              rows0, rows1, ob0, ob1, gsem0, gsem1, ssem0, ssem1):
    wid = lax.axis_index("s") * NC + lax.axis_index("c")
    nchunk = out_hbm.shape[1]
    nloop = nchunk // 2

    pltpu.sync_copy(x_hbm.at[wid], idx_v)
    _gather_start(table_hbm, idx_v, 0, idxs0, rows0, gsem0)

    def pair(i, carry):
        c0 = 2 * i

        @pl.when(i > 0)
        def _():
            pltpu.make_async_copy(ob1, out_hbm.at[wid, c0], ssem1).wait()

        _gather_start(table_hbm, idx_v, c0 + 1, idxs1, rows1, gsem1)
        _gather_wait(table_hbm, rows0, gsem0)

        @pl.when(i > 0)
        def _():
            pltpu.make_async_copy(ob0, out_hbm.at[wid, c0], ssem0).wait()

        _extract(idx_v, c0, rows0, ob0)
        pltpu.async_copy(ob0, out_hbm.at[wid, c0], ssem0)

        @pl.when(i < nloop - 1)
        def _():
            _gather_start(table_hbm, idx_v, c0 + 2, idxs0, rows0, gsem0)

        _gather_wait(table_hbm, rows1, gsem1)
        _extract(idx_v, c0 + 1, rows1, ob1)
        pltpu.async_copy(ob1, out_hbm.at[wid, c0 + 1], ssem1)
        return carry

    lax.fori_loop(0, nloop, pair, jnp.int32(0))
    pltpu.make_async_copy(ob0, out_hbm.at[wid, nchunk - 2], ssem0).wait()
    pltpu.make_async_copy(ob1, out_hbm.at[wid, nchunk - 1], ssem1).wait()


@jax.jit
def kernel(x, table):
    b, l = x.shape
    voc, dim = table.shape
    total = b * l
    nchunk = total // (NW * CHUNK)
    xr = x.astype(jnp.int32).reshape(NW, nchunk * CHUNK)
    t2 = table.reshape(voc // 2, 2 * dim)
    mesh = plsc.VectorSubcoreMesh(
        core_axis_name="c", subcore_axis_name="s",
        num_cores=NC, num_subcores=NS,
    )
    out = pl.kernel(
        _emb_body,
        out_type=jax.ShapeDtypeStruct((NW, nchunk, CHUNK, DIM), jnp.float32),
        mesh=mesh,
        compiler_params=pltpu.CompilerParams(needs_layout_passes=False),
        scratch_types=[
            pltpu.VMEM((nchunk * CHUNK,), jnp.int32),
            pltpu.VMEM((NSUB, SUB), jnp.int32),
            pltpu.VMEM((NSUB, SUB), jnp.int32),
            pltpu.VMEM((CHUNK, 2 * DIM), jnp.float32),
            pltpu.VMEM((CHUNK, 2 * DIM), jnp.float32),
            pltpu.VMEM((CHUNK, DIM), jnp.float32),
            pltpu.VMEM((CHUNK, DIM), jnp.float32),
            pltpu.SemaphoreType.DMA,
            pltpu.SemaphoreType.DMA,
            pltpu.SemaphoreType.DMA,
            pltpu.SemaphoreType.DMA,
        ],
    )(xr, t2)
    return out.reshape(b, l, DIM)
